# baseline (speedup 1.0000x reference)
import functools

import jax
import jax.numpy as jnp
from jax import lax
from jax.experimental import pallas as pl
from jax.experimental.pallas import tpu as pltpu
from jax.experimental.pallas import tpu_sc as plsc

D_MODEL = 768
D_FF = 3072
NUM_EXPERTS = 8
TOP_K = 2
N_TOKENS = 2048
N_SLOTS = N_TOKENS * TOP_K

BLK = 256
NB = N_SLOTS // BLK + NUM_EXPERTS
P_ROWS = NB * BLK

_SC_NW = 32
_D_TOK = N_TOKENS // _SC_NW

_SC_INFO = plsc.get_sparse_core_info()
NC = _SC_INFO.num_cores
NS = _SC_INFO.num_subcores
NW = NC * NS


def _router_kernel(x_ref, wr_ref, exp_ref, w_ref):
    logits = jnp.dot(x_ref[...], wr_ref[...], preferred_element_type=jnp.float32)
    n = logits.shape[0]
    io = lax.broadcasted_iota(jnp.int32, (n, NUM_EXPERTS), 1)
    m1 = jnp.max(logits, axis=1, keepdims=True)
    i1 = jnp.min(jnp.where(logits == m1, io, NUM_EXPERTS), axis=1, keepdims=True)
    masked = jnp.where(io == i1, -jnp.inf, logits)
    m2 = jnp.max(masked, axis=1, keepdims=True)
    i2 = jnp.min(jnp.where(masked == m2, io, NUM_EXPERTS), axis=1, keepdims=True)
    d = jnp.exp(m2 - m1)
    w1 = 1.0 / (1.0 + d)
    exp_ref[...] = jnp.concatenate([i1, i2], axis=1)
    w_ref[...] = jnp.concatenate(
        [jnp.broadcast_to(w1, (n, 16)), jnp.broadcast_to(1.0 - w1, (n, 16))],
        axis=1)


def _router(x_flat, Wr):
    return pl.pallas_call(
        _router_kernel,
        out_shape=(
            jax.ShapeDtypeStruct((N_TOKENS, TOP_K), jnp.int32),
            jax.ShapeDtypeStruct((N_TOKENS, TOP_K * 16), jnp.float32),
        ),
    )(x_flat, Wr)


def _dispatch(experts):
    e_flat = experts.reshape(-1)
    onehot = (e_flat[:, None] == jnp.arange(NUM_EXPERTS)[None, :]).astype(jnp.int32)
    cum = jnp.cumsum(onehot, axis=0)
    cnt = cum[-1]
    pc = ((cnt + BLK - 1) // BLK) * BLK
    poff = jnp.cumsum(pc) - pc
    p_i = jnp.sum(onehot * (poff[None, :] + cum), axis=1) - 1
    pos3 = p_i.astype(jnp.int32).reshape(NW, _D_TOK, TOP_K).transpose(0, 2, 1)
    bstart = jnp.arange(NB, dtype=jnp.int32) * BLK
    be = -jnp.ones((NB,), jnp.int32)
    for e in range(NUM_EXPERTS):
        be = jnp.where((bstart >= poff[e]) & (bstart < poff[e] + pc[e]), e, be)
    return pos3, be


@functools.partial(
    pl.kernel,
    mesh=plsc.VectorSubcoreMesh(core_axis_name="c", subcore_axis_name="s"),
    out_type=jax.ShapeDtypeStruct((P_ROWS, D_MODEL), jnp.float32),
    scratch_types=[
        pltpu.VMEM((TOP_K, _D_TOK), jnp.int32),
        pltpu.VMEM((_D_TOK, D_MODEL), jnp.float32),
        pltpu.SemaphoreType.DMA,
        pltpu.SemaphoreType.DMA,
    ],
)
def _sc_dispatch_rows(x_hbm, pos3_hbm, xs_hbm, idx_v, buf_v, sem_a, sem_b):
    wid = lax.axis_index("s") * NC + lax.axis_index("c")
    pltpu.sync_copy(pos3_hbm.at[wid], idx_v)
    pltpu.sync_copy(x_hbm.at[pl.ds(wid * _D_TOK, _D_TOK)], buf_v)
    a = pltpu.async_copy(buf_v, xs_hbm.at[idx_v.at[0]], sem_a)
    b = pltpu.async_copy(buf_v, xs_hbm.at[idx_v.at[1]], sem_b)
    a.wait()
    b.wait()


FB = D_FF // 4


def _ffn_kernel(be_ref, xs_ref, w1_ref, b1_ref, w2_ref, b2_ref, out_ref):
    b = pl.program_id(0)
    f = pl.program_id(1)

    @pl.when(jnp.logical_and(be_ref[b] >= 0, f == 0))
    def _():
        out_ref[...] = jnp.broadcast_to(b2_ref[0], out_ref.shape)

    @pl.when(be_ref[b] >= 0)
    def _():
        h = jnp.dot(xs_ref[...], w1_ref[0], preferred_element_type=jnp.float32)
        h = h + b1_ref[0]
        h = 0.5 * h * (1.0 + lax.erf(h * (2.0 ** -0.5)))
        out_ref[...] += jnp.dot(h, w2_ref[0], preferred_element_type=jnp.float32)

    @pl.when(jnp.logical_and(be_ref[b] < 0, f == 0))
    def _():
        out_ref[...] = jnp.zeros_like(out_ref)


def _ffn(xs, W1, b1, W2, b2, be):
    def eof(b, f, be_ref):
        return jnp.where(be_ref[b] < 0, NUM_EXPERTS - 1, be_ref[b])

    grid_spec = pltpu.PrefetchScalarGridSpec(
        num_scalar_prefetch=1,
        grid=(NB, D_FF // FB),
        in_specs=[
            pl.BlockSpec((BLK, D_MODEL), lambda b, f, be_ref: (b, 0)),
            pl.BlockSpec((1, D_MODEL, FB), lambda b, f, be_ref: (eof(b, f, be_ref), 0, f)),
            pl.BlockSpec((1, 1, FB), lambda b, f, be_ref: (eof(b, f, be_ref), 0, f)),
            pl.BlockSpec((1, FB, D_MODEL), lambda b, f, be_ref: (eof(b, f, be_ref), f, 0)),
            pl.BlockSpec((1, 1, D_MODEL), lambda b, f, be_ref: (eof(b, f, be_ref), 0, 0)),
        ],
        out_specs=pl.BlockSpec((BLK, D_MODEL), lambda b, f, be_ref: (b, 0)),
    )
    return pl.pallas_call(
        _ffn_kernel,
        grid_spec=grid_spec,
        out_shape=jax.ShapeDtypeStruct((P_ROWS, D_MODEL), jnp.float32),
    )(be, xs, W1, b1.reshape(NUM_EXPERTS, 1, D_FF), W2,
      b2.reshape(NUM_EXPERTS, 1, D_MODEL))


_C_LANES = D_MODEL // 16


@functools.partial(
    pl.kernel,
    mesh=plsc.VectorSubcoreMesh(core_axis_name="c", subcore_axis_name="s"),
    out_type=jax.ShapeDtypeStruct((N_TOKENS, D_MODEL), jnp.float32),
    scratch_types=[
        pltpu.VMEM((TOP_K, _D_TOK), jnp.int32),
        pltpu.VMEM((_D_TOK, TOP_K * 16), jnp.float32),
        pltpu.VMEM((_D_TOK, D_MODEL), jnp.float32),
        pltpu.VMEM((_D_TOK, D_MODEL), jnp.float32),
        pltpu.SemaphoreType.DMA,
        pltpu.SemaphoreType.DMA,
    ],
)
def _sc_combine(ys_hbm, pos3_hbm, w_hbm, out_hbm, idx_v, w_v, buf_a, buf_b,
                sem_a, sem_b):
    wid = lax.axis_index("s") * NC + lax.axis_index("c")
    pltpu.sync_copy(pos3_hbm.at[wid], idx_v)
    pltpu.sync_copy(w_hbm.at[pl.ds(wid * _D_TOK, _D_TOK)], w_v)
    a = pltpu.async_copy(ys_hbm.at[idx_v.at[0]], buf_a, sem_a)
    b = pltpu.async_copy(ys_hbm.at[idx_v.at[1]], buf_b, sem_b)
    a.wait()
    b.wait()

    def body(i, _):
        w0 = w_v[i, pl.ds(0, 16)]
        w1 = w_v[i, pl.ds(16, 16)]
        for j in range(_C_LANES):
            s = pl.ds(j * 16, 16)
            buf_a[i, s] = buf_a[i, s] * w0 + buf_b[i, s] * w1
        return 0

    lax.fori_loop(0, _D_TOK, body, 0)
    pltpu.sync_copy(buf_a, out_hbm.at[pl.ds(wid * _D_TOK, _D_TOK)])


def kernel(x, Wr, W1, b1, W2, b2):
    Bv, Tv, C = x.shape
    x_flat = x.reshape(-1, C)
    experts, weights = _router(x_flat, Wr)
    pos3, be = _dispatch(experts)
    xs = _sc_dispatch_rows(x_flat, pos3)
    ys = _ffn(xs, W1, b1, W2, b2, be)
    out = _sc_combine(ys, pos3, weights)
    return out.reshape(Bv, Tv, C)


# speedup vs baseline: 1.4455x; 1.4455x over previous
import functools

import jax
import jax.numpy as jnp
from jax import lax
from jax.experimental import pallas as pl
from jax.experimental.pallas import tpu as pltpu
from jax.experimental.pallas import tpu_sc as plsc

D_MODEL = 768
D_FF = 3072
NUM_EXPERTS = 8
TOP_K = 2
N_TOKENS = 2048
N_SLOTS = N_TOKENS * TOP_K

BLK = 256
NB = N_SLOTS // BLK + NUM_EXPERTS
P_ROWS = NB * BLK

_SC_NW = 32
_D_TOK = N_TOKENS // _SC_NW

_SC_INFO = plsc.get_sparse_core_info()
NC = _SC_INFO.num_cores
NS = _SC_INFO.num_subcores
NW = NC * NS


def _router_kernel(x_ref, wr_ref, exp_ref, w_ref):
    logits = jnp.dot(x_ref[...], wr_ref[...], preferred_element_type=jnp.float32)
    n = logits.shape[0]
    io = lax.broadcasted_iota(jnp.int32, (n, NUM_EXPERTS), 1)
    m1 = jnp.max(logits, axis=1, keepdims=True)
    i1 = jnp.min(jnp.where(logits == m1, io, NUM_EXPERTS), axis=1, keepdims=True)
    masked = jnp.where(io == i1, -jnp.inf, logits)
    m2 = jnp.max(masked, axis=1, keepdims=True)
    i2 = jnp.min(jnp.where(masked == m2, io, NUM_EXPERTS), axis=1, keepdims=True)
    d = jnp.exp(m2 - m1)
    w1 = 1.0 / (1.0 + d)
    exp_ref[...] = jnp.concatenate([i1, i2], axis=1)
    w_ref[...] = jnp.concatenate(
        [jnp.broadcast_to(w1, (n, 16)), jnp.broadcast_to(1.0 - w1, (n, 16))],
        axis=1)


def _router(x_flat, Wr):
    return pl.pallas_call(
        _router_kernel,
        out_shape=(
            jax.ShapeDtypeStruct((N_TOKENS, TOP_K), jnp.int32),
            jax.ShapeDtypeStruct((N_TOKENS, TOP_K * 16), jnp.float32),
        ),
    )(x_flat, Wr)


def _dispatch(experts):
    e_flat = experts.reshape(-1)
    onehot = (e_flat[:, None] == jnp.arange(NUM_EXPERTS)[None, :]).astype(jnp.int32)
    cum = jnp.cumsum(onehot, axis=0)
    cnt = cum[-1]
    pc = ((cnt + BLK - 1) // BLK) * BLK
    poff = jnp.cumsum(pc) - pc
    p_i = jnp.sum(onehot * (poff[None, :] + cum), axis=1) - 1
    pos3 = p_i.astype(jnp.int32).reshape(NW, _D_TOK, TOP_K).transpose(0, 2, 1)
    bstart = jnp.arange(NB, dtype=jnp.int32) * BLK
    be = -jnp.ones((NB,), jnp.int32)
    for e in range(NUM_EXPERTS):
        be = jnp.where((bstart >= poff[e]) & (bstart < poff[e] + pc[e]), e, be)
    return pos3, be


@functools.partial(
    pl.kernel,
    mesh=plsc.VectorSubcoreMesh(core_axis_name="c", subcore_axis_name="s"),
    out_type=jax.ShapeDtypeStruct((P_ROWS, D_MODEL), jnp.float32),
    scratch_types=[
        pltpu.VMEM((TOP_K, _D_TOK), jnp.int32),
        pltpu.VMEM((_D_TOK, D_MODEL), jnp.float32),
        pltpu.SemaphoreType.DMA,
        pltpu.SemaphoreType.DMA,
    ],
)
def _sc_dispatch_rows(x_hbm, pos3_hbm, xs_hbm, idx_v, buf_v, sem_a, sem_b):
    wid = lax.axis_index("s") * NC + lax.axis_index("c")
    pltpu.sync_copy(pos3_hbm.at[wid], idx_v)
    pltpu.sync_copy(x_hbm.at[pl.ds(wid * _D_TOK, _D_TOK)], buf_v)
    a = pltpu.async_copy(buf_v, xs_hbm.at[idx_v.at[0]], sem_a)
    b = pltpu.async_copy(buf_v, xs_hbm.at[idx_v.at[1]], sem_b)
    a.wait()
    b.wait()


def _ffn_kernel(be_ref, xs_ref, w1_ref, b1_ref, w2_ref, b2_ref, out_ref):
    b = pl.program_id(0)

    @pl.when(be_ref[b] >= 0)
    def _():
        h = jnp.dot(xs_ref[...], w1_ref[0], preferred_element_type=jnp.float32)
        h = h + b1_ref[0]
        h = 0.5 * h * (1.0 + lax.erf(h * (2.0 ** -0.5)))
        y = jnp.dot(h, w2_ref[0], preferred_element_type=jnp.float32)
        out_ref[...] = y + b2_ref[0]

    @pl.when(be_ref[b] < 0)
    def _():
        out_ref[...] = jnp.zeros_like(out_ref)


def _ffn(xs, W1, b1, W2, b2, be):
    def emap(b, be_ref):
        return (jnp.where(be_ref[b] < 0, NUM_EXPERTS - 1, be_ref[b]), 0, 0)

    grid_spec = pltpu.PrefetchScalarGridSpec(
        num_scalar_prefetch=1,
        grid=(NB,),
        in_specs=[
            pl.BlockSpec((BLK, D_MODEL), lambda b, be_ref: (b, 0)),
            pl.BlockSpec((1, D_MODEL, D_FF), emap),
            pl.BlockSpec((1, 1, D_FF), emap),
            pl.BlockSpec((1, D_FF, D_MODEL), emap),
            pl.BlockSpec((1, 1, D_MODEL), emap),
        ],
        out_specs=pl.BlockSpec((BLK, D_MODEL), lambda b, be_ref: (b, 0)),
    )
    return pl.pallas_call(
        _ffn_kernel,
        grid_spec=grid_spec,
        out_shape=jax.ShapeDtypeStruct((P_ROWS, D_MODEL), jnp.float32),
    )(be, xs, W1, b1.reshape(NUM_EXPERTS, 1, D_FF), W2,
      b2.reshape(NUM_EXPERTS, 1, D_MODEL))


_C_LANES = D_MODEL // 16


@functools.partial(
    pl.kernel,
    mesh=plsc.VectorSubcoreMesh(core_axis_name="c", subcore_axis_name="s"),
    out_type=jax.ShapeDtypeStruct((N_TOKENS, D_MODEL), jnp.float32),
    scratch_types=[
        pltpu.VMEM((TOP_K, _D_TOK), jnp.int32),
        pltpu.VMEM((_D_TOK, TOP_K * 16), jnp.float32),
        pltpu.VMEM((_D_TOK, D_MODEL), jnp.float32),
        pltpu.VMEM((_D_TOK, D_MODEL), jnp.float32),
        pltpu.SemaphoreType.DMA,
        pltpu.SemaphoreType.DMA,
    ],
)
def _sc_combine(ys_hbm, pos3_hbm, w_hbm, out_hbm, idx_v, w_v, buf_a, buf_b,
                sem_a, sem_b):
    wid = lax.axis_index("s") * NC + lax.axis_index("c")
    pltpu.sync_copy(pos3_hbm.at[wid], idx_v)
    pltpu.sync_copy(w_hbm.at[pl.ds(wid * _D_TOK, _D_TOK)], w_v)
    a = pltpu.async_copy(ys_hbm.at[idx_v.at[0]], buf_a, sem_a)
    b = pltpu.async_copy(ys_hbm.at[idx_v.at[1]], buf_b, sem_b)
    a.wait()
    b.wait()

    def body(i, _):
        w0 = w_v[i, pl.ds(0, 16)]
        w1 = w_v[i, pl.ds(16, 16)]
        for j in range(_C_LANES):
            s = pl.ds(j * 16, 16)
            buf_a[i, s] = buf_a[i, s] * w0 + buf_b[i, s] * w1
        return 0

    lax.fori_loop(0, _D_TOK, body, 0)
    pltpu.sync_copy(buf_a, out_hbm.at[pl.ds(wid * _D_TOK, _D_TOK)])


def kernel(x, Wr, W1, b1, W2, b2):
    Bv, Tv, C = x.shape
    x_flat = x.reshape(-1, C)
    experts, weights = _router(x_flat, Wr)
    pos3, be = _dispatch(experts)
    xs = _sc_dispatch_rows(x_flat, pos3)
    ys = _ffn(xs, W1, b1, W2, b2, be)
    out = _sc_combine(ys, pos3, weights)
    return out.reshape(Bv, Tv, C)


# speedup vs baseline: 1.4610x; 1.0107x over previous
import functools

import jax
import jax.numpy as jnp
from jax import lax
from jax.experimental import pallas as pl
from jax.experimental.pallas import tpu as pltpu
from jax.experimental.pallas import tpu_sc as plsc

D_MODEL = 768
D_FF = 3072
NUM_EXPERTS = 8
TOP_K = 2
N_TOKENS = 2048
N_SLOTS = N_TOKENS * TOP_K

BLK = 256
NB = N_SLOTS // BLK + NUM_EXPERTS
P_ROWS = NB * BLK

_SC_NW = 32
_D_TOK = N_TOKENS // _SC_NW

_SC_INFO = plsc.get_sparse_core_info()
NC = _SC_INFO.num_cores
NS = _SC_INFO.num_subcores
NW = NC * NS


def _router_kernel(x_ref, wr_ref, exp_ref, w_ref):
    logits = jnp.dot(x_ref[...], wr_ref[...], preferred_element_type=jnp.float32)
    n = logits.shape[0]
    io = lax.broadcasted_iota(jnp.int32, (n, NUM_EXPERTS), 1)
    m1 = jnp.max(logits, axis=1, keepdims=True)
    i1 = jnp.min(jnp.where(logits == m1, io, NUM_EXPERTS), axis=1, keepdims=True)
    masked = jnp.where(io == i1, -jnp.inf, logits)
    m2 = jnp.max(masked, axis=1, keepdims=True)
    i2 = jnp.min(jnp.where(masked == m2, io, NUM_EXPERTS), axis=1, keepdims=True)
    d = jnp.exp(m2 - m1)
    w1 = 1.0 / (1.0 + d)
    exp_ref[...] = jnp.concatenate([i1, i2], axis=1)
    w_ref[...] = jnp.concatenate(
        [jnp.broadcast_to(w1, (n, 16)), jnp.broadcast_to(1.0 - w1, (n, 16))],
        axis=1)


def _router(x_flat, Wr):
    return pl.pallas_call(
        _router_kernel,
        out_shape=(
            jax.ShapeDtypeStruct((N_TOKENS, TOP_K), jnp.int32),
            jax.ShapeDtypeStruct((N_TOKENS, TOP_K * 16), jnp.float32),
        ),
    )(x_flat, Wr)


def _dispatch(experts):
    e_flat = experts.reshape(-1)
    onehot = (e_flat[:, None] == jnp.arange(NUM_EXPERTS)[None, :]).astype(jnp.int32)
    cum = jnp.cumsum(onehot, axis=0)
    cnt = cum[-1]
    pc = ((cnt + BLK - 1) // BLK) * BLK
    poff = jnp.cumsum(pc) - pc
    p_i = jnp.sum(onehot * (poff[None, :] + cum), axis=1) - 1
    pos3 = p_i.astype(jnp.int32).reshape(NW, _D_TOK, TOP_K).transpose(0, 2, 1)
    bstart = jnp.arange(NB, dtype=jnp.int32) * BLK
    be = -jnp.ones((NB,), jnp.int32)
    for e in range(NUM_EXPERTS):
        be = jnp.where((bstart >= poff[e]) & (bstart < poff[e] + pc[e]), e, be)
    return pos3, be


@functools.partial(
    pl.kernel,
    mesh=plsc.VectorSubcoreMesh(core_axis_name="c", subcore_axis_name="s"),
    out_type=jax.ShapeDtypeStruct((P_ROWS, D_MODEL), jnp.float32),
    scratch_types=[
        pltpu.VMEM((TOP_K, _D_TOK), jnp.int32),
        pltpu.VMEM((_D_TOK, D_MODEL), jnp.float32),
        pltpu.SemaphoreType.DMA,
        pltpu.SemaphoreType.DMA,
    ],
)
def _sc_dispatch_rows(x_hbm, pos3_hbm, xs_hbm, idx_v, buf_v, sem_a, sem_b):
    wid = lax.axis_index("s") * NC + lax.axis_index("c")
    pltpu.sync_copy(pos3_hbm.at[wid], idx_v)
    pltpu.sync_copy(x_hbm.at[pl.ds(wid * _D_TOK, _D_TOK)], buf_v)
    a = pltpu.async_copy(buf_v, xs_hbm.at[idx_v.at[0]], sem_a)
    b = pltpu.async_copy(buf_v, xs_hbm.at[idx_v.at[1]], sem_b)
    a.wait()
    b.wait()


def _ffn_kernel(be_ref, xs_ref, w1_ref, b1_ref, w2_ref, b2_ref, out_ref):
    b = pl.program_id(0)

    @pl.when(be_ref[b] >= 0)
    def _():
        h = jnp.dot(xs_ref[...], w1_ref[0], preferred_element_type=jnp.float32)
        h = h + b1_ref[0]
        h = 0.5 * h * (1.0 + lax.erf(h * (2.0 ** -0.5)))
        y = jnp.dot(h, w2_ref[0], preferred_element_type=jnp.float32)
        out_ref[...] = y + b2_ref[0]

    @pl.when(be_ref[b] < 0)
    def _():
        out_ref[...] = jnp.zeros_like(out_ref)


def _ffn(xs, W1, b1, W2, b2, be):
    def emap(b, be_ref):
        return (jnp.where(be_ref[b] < 0, NUM_EXPERTS - 1, be_ref[b]), 0, 0)

    grid_spec = pltpu.PrefetchScalarGridSpec(
        num_scalar_prefetch=1,
        grid=(NB,),
        in_specs=[
            pl.BlockSpec((BLK, D_MODEL),
                         lambda b, be_ref: (jnp.where(be_ref[b] < 0, 0, b), 0)),
            pl.BlockSpec((1, D_MODEL, D_FF), emap),
            pl.BlockSpec((1, 1, D_FF), emap),
            pl.BlockSpec((1, D_FF, D_MODEL), emap),
            pl.BlockSpec((1, 1, D_MODEL), emap),
        ],
        out_specs=pl.BlockSpec((BLK, D_MODEL), lambda b, be_ref: (b, 0)),
    )
    return pl.pallas_call(
        _ffn_kernel,
        grid_spec=grid_spec,
        out_shape=jax.ShapeDtypeStruct((P_ROWS, D_MODEL), jnp.float32),
    )(be, xs, W1, b1.reshape(NUM_EXPERTS, 1, D_FF), W2,
      b2.reshape(NUM_EXPERTS, 1, D_MODEL))


_C_LANES = D_MODEL // 16


@functools.partial(
    pl.kernel,
    mesh=plsc.VectorSubcoreMesh(core_axis_name="c", subcore_axis_name="s"),
    out_type=jax.ShapeDtypeStruct((N_TOKENS, D_MODEL), jnp.float32),
    scratch_types=[
        pltpu.VMEM((TOP_K, _D_TOK), jnp.int32),
        pltpu.VMEM((_D_TOK, TOP_K * 16), jnp.float32),
        pltpu.VMEM((_D_TOK, D_MODEL), jnp.float32),
        pltpu.VMEM((_D_TOK, D_MODEL), jnp.float32),
        pltpu.SemaphoreType.DMA,
        pltpu.SemaphoreType.DMA,
    ],
)
def _sc_combine(ys_hbm, pos3_hbm, w_hbm, out_hbm, idx_v, w_v, buf_a, buf_b,
                sem_a, sem_b):
    wid = lax.axis_index("s") * NC + lax.axis_index("c")
    pltpu.sync_copy(pos3_hbm.at[wid], idx_v)
    pltpu.sync_copy(w_hbm.at[pl.ds(wid * _D_TOK, _D_TOK)], w_v)
    a = pltpu.async_copy(ys_hbm.at[idx_v.at[0]], buf_a, sem_a)
    b = pltpu.async_copy(ys_hbm.at[idx_v.at[1]], buf_b, sem_b)
    a.wait()
    b.wait()

    def body(i, _):
        w0 = w_v[i, pl.ds(0, 16)]
        w1 = w_v[i, pl.ds(16, 16)]
        for j in range(_C_LANES):
            s = pl.ds(j * 16, 16)
            buf_a[i, s] = buf_a[i, s] * w0 + buf_b[i, s] * w1
        return 0

    lax.fori_loop(0, _D_TOK, body, 0)
    pltpu.sync_copy(buf_a, out_hbm.at[pl.ds(wid * _D_TOK, _D_TOK)])


def kernel(x, Wr, W1, b1, W2, b2):
    Bv, Tv, C = x.shape
    x_flat = x.reshape(-1, C)
    experts, weights = _router(x_flat, Wr)
    pos3, be = _dispatch(experts)
    xs = _sc_dispatch_rows(x_flat, pos3)
    ys = _ffn(xs, W1, b1, W2, b2, be)
    out = _sc_combine(ys, pos3, weights)
    return out.reshape(Bv, Tv, C)
